# baseline (speedup 1.0000x reference)
import functools

import jax
import jax.numpy as jnp
from jax import lax
from jax.experimental import pallas as pl
from jax.experimental.pallas import tpu as pltpu
from jax.experimental.pallas import tpu_sc as plsc

DEPTH = 1000
BATCH = 4096
HIST = 20
NC = 2
NS = 16
L = 16
NW = NC * NS
BPW = BATCH // NW
BGROUPS = BPW // L
DCH = 200
QN = DEPTH // DCH
NCHUNK = HIST * QN

_mesh = plsc.VectorSubcoreMesh(core_axis_name="c", subcore_axis_name="s")


@functools.partial(
    pl.kernel,
    mesh=_mesh,
    out_type=jax.ShapeDtypeStruct((HIST, DEPTH, BATCH), jnp.float32),
    scratch_types=[
        pltpu.VMEM((BPW, HIST), jnp.int32),
        pltpu.VMEM((HIST * BPW,), jnp.int32),
        pltpu.VMEM((1, DCH, BPW), jnp.float32),
        pltpu.VMEM((1, DCH, BPW), jnp.float32),
        pltpu.SemaphoreType.DMA,
        pltpu.SemaphoreType.DMA,
    ],
    compiler_params=pltpu.CompilerParams(needs_layout_passes=False),
)
def _sc_onehot(idx_hbm, out_hbm, idx_v, idx_t, buf0, buf1, sem0, sem1):
    bufs = (buf0, buf1)
    sems = (sem0, sem1)
    wid = lax.axis_index("s") * NC + lax.axis_index("c")
    base_b = wid * BPW
    pltpu.sync_copy(idx_hbm.at[pl.ds(base_b, BPW)], idx_v)

    zeros16 = jnp.zeros((L,), jnp.float32)
    ones16 = jnp.full((L,), 1.0, jnp.float32)
    iota16 = lax.iota(jnp.int32, L)

    for h in range(HIST):
        h16 = jnp.broadcast_to(h, (L,))
        for g in range(BGROUPS):
            b16 = g * L + iota16
            vals = plsc.load_gather(idx_v, [b16, h16])
            idx_t[pl.ds(h * BPW + g * L, L)] = vals

    def zbody(r, carry):
        for buf in bufs:
            for g in range(BGROUPS):
                buf[0, r, pl.ds(g * L, L)] = zeros16
        return carry

    lax.fori_loop(0, DCH, zbody, 0)

    zero16i = jnp.zeros((L,), jnp.int32)

    def scatter_chunk(h, q, buf, value_vec):
        d0 = q * DCH
        for g in range(BGROUPS):
            b16 = g * L + iota16
            d16 = idx_t[pl.ds(h * BPW + g * L, L)]
            dloc = d16 - d0
            mask = (dloc >= 0) & (dloc < DCH)
            plsc.store_scatter(buf, [zero16i, dloc, b16], value_vec, mask=mask)

    def start_dma(h, q, buf, sem):
        dst = out_hbm.at[pl.ds(h, 1), pl.ds(q * DCH, DCH), pl.ds(base_b, BPW)]
        pltpu.async_copy(buf, dst, sem)

    def drain(h, q, buf, sem):
        dst = out_hbm.at[pl.ds(h, 1), pl.ds(q * DCH, DCH), pl.ds(base_b, BPW)]
        pltpu.make_async_copy(buf, dst, sem).wait()

    for s in range(2):
        scatter_chunk(s // QN, s % QN, bufs[s], ones16)
        start_dma(s // QN, s % QN, bufs[s], sems[s])

    def pair_body(i, carry):
        c0 = 2 + 2 * i
        for s in range(2):
            c = c0 + s
            hp, qp = (c - 2) // QN, (c - 2) % QN
            h, q = c // QN, c % QN
            drain(hp, qp, bufs[s], sems[s])
            scatter_chunk(hp, qp, bufs[s], zeros16)
            scatter_chunk(h, q, bufs[s], ones16)
            start_dma(h, q, bufs[s], sems[s])
        return carry

    lax.fori_loop(0, (NCHUNK - 2) // 2, pair_body, 0)

    for s in range(2):
        c = NCHUNK - 2 + s
        drain(c // QN, c % QN, bufs[s], sems[s])


def kernel(input, emb_weight):
    del emb_weight
    out_t = _sc_onehot(input)
    return out_t.transpose(2, 0, 1)


# speedup vs baseline: 1.0131x; 1.0131x over previous
import functools

import jax
import jax.numpy as jnp
from jax import lax
from jax.experimental import pallas as pl
from jax.experimental.pallas import tpu as pltpu
from jax.experimental.pallas import tpu_sc as plsc

DEPTH = 1000
BATCH = 4096
HIST = 20
NC = 2
NS = 16
L = 16
NW = NC * NS
BPW = BATCH // NW
BGROUPS = BPW // L
DCH = 200
QN = DEPTH // DCH
NCHUNK = HIST * QN

_mesh = plsc.VectorSubcoreMesh(core_axis_name="c", subcore_axis_name="s")


@functools.partial(
    pl.kernel,
    mesh=_mesh,
    out_type=jax.ShapeDtypeStruct((HIST, DEPTH, BATCH), jnp.float32),
    scratch_types=[
        pltpu.VMEM((BPW * HIST,), jnp.int32),
        pltpu.VMEM((HIST * BPW,), jnp.int32),
        pltpu.VMEM((1, DCH, BPW), jnp.float32),
        pltpu.VMEM((1, DCH, BPW), jnp.float32),
        pltpu.SemaphoreType.DMA,
        pltpu.SemaphoreType.DMA,
    ],
    compiler_params=pltpu.CompilerParams(needs_layout_passes=False),
)
def _sc_onehot(idx_hbm, out_hbm, idx_v, idx_t, buf0, buf1, sem0, sem1):
    bufs = (buf0, buf1)
    sems = (sem0, sem1)
    wid = lax.axis_index("s") * NC + lax.axis_index("c")
    base_b = wid * BPW
    pltpu.sync_copy(idx_hbm.at[pl.ds(base_b * HIST, BPW * HIST)], idx_v)

    zeros16 = jnp.zeros((L,), jnp.float32)
    ones16 = jnp.full((L,), 1.0, jnp.float32)
    iota16 = lax.iota(jnp.int32, L)

    for h in range(HIST):
        for g in range(BGROUPS):
            b16 = g * L + iota16
            vals = plsc.load_gather(idx_v, [b16 * HIST + h])
            idx_t[pl.ds(h * BPW + g * L, L)] = vals

    def zbody(r, carry):
        for buf in bufs:
            for g in range(BGROUPS):
                buf[0, r, pl.ds(g * L, L)] = zeros16
        return carry

    lax.fori_loop(0, DCH, zbody, 0)

    zero16i = jnp.zeros((L,), jnp.int32)

    def scatter_chunk(h, q, buf, value_vec):
        d0 = q * DCH
        for g in range(BGROUPS):
            b16 = g * L + iota16
            d16 = idx_t[pl.ds(h * BPW + g * L, L)]
            dloc = d16 - d0
            mask = (dloc >= 0) & (dloc < DCH)
            plsc.store_scatter(buf, [zero16i, dloc, b16], value_vec, mask=mask)

    def start_dma(h, q, buf, sem):
        dst = out_hbm.at[pl.ds(h, 1), pl.ds(q * DCH, DCH), pl.ds(base_b, BPW)]
        pltpu.async_copy(buf, dst, sem)

    def drain(h, q, buf, sem):
        dst = out_hbm.at[pl.ds(h, 1), pl.ds(q * DCH, DCH), pl.ds(base_b, BPW)]
        pltpu.make_async_copy(buf, dst, sem).wait()

    for s in range(2):
        scatter_chunk(s // QN, s % QN, bufs[s], ones16)
        start_dma(s // QN, s % QN, bufs[s], sems[s])

    def pair_body(i, carry):
        c0 = 2 + 2 * i
        for s in range(2):
            c = c0 + s
            hp, qp = (c - 2) // QN, (c - 2) % QN
            h, q = c // QN, c % QN
            drain(hp, qp, bufs[s], sems[s])
            scatter_chunk(hp, qp, bufs[s], zeros16)
            scatter_chunk(h, q, bufs[s], ones16)
            start_dma(h, q, bufs[s], sems[s])
        return carry

    lax.fori_loop(0, (NCHUNK - 2) // 2, pair_body, 0)

    for s in range(2):
        c = NCHUNK - 2 + s
        drain(c // QN, c % QN, bufs[s], sems[s])


def kernel(input, emb_weight):
    del emb_weight
    out_t = _sc_onehot(input.reshape(-1))
    return out_t.transpose(2, 0, 1)
